# Initial kernel scaffold; baseline (speedup 1.0000x reference)
import jax
import jax.numpy as jnp
from jax.experimental import pallas as pl


def kernel(t, beta, alpha_cum):
    raise NotImplementedError("write your pallas kernel here")


# speedup vs baseline: 8.3247x; 8.3247x over previous
import functools

import jax
import jax.numpy as jnp
from jax import lax
from jax.experimental import pallas as pl
from jax.experimental.pallas import tpu as pltpu
from jax.experimental.pallas import tpu_sc as plsc

_B = 16384
_T = 1000
_TPAD = 1024

_info = plsc.get_sparse_core_info()
_NC, _NS, _L = _info.num_cores, _info.num_subcores, _info.num_lanes
_NW = _NC * _NS
_BPW = _B // _NW


def _gather_body(t_hbm, beta_hbm, alpha_hbm, out_b_hbm, out_a_hbm,
                 idx_v, beta_v, alpha_v, outb_v, outa_v):
    wid = lax.axis_index("s") * _NC + lax.axis_index("c")
    base = wid * _BPW
    pltpu.sync_copy(beta_hbm, beta_v)
    pltpu.sync_copy(alpha_hbm, alpha_v)
    pltpu.sync_copy(t_hbm.at[pl.ds(base, _BPW)], idx_v)
    for i in range(_BPW // _L):
        sl = pl.ds(i * _L, _L)
        idx = idx_v[sl]
        outb_v[sl] = plsc.load_gather(beta_v, [idx])
        outa_v[sl] = plsc.load_gather(alpha_v, [idx])
    pltpu.sync_copy(outb_v, out_b_hbm.at[pl.ds(base, _BPW)])
    pltpu.sync_copy(outa_v, out_a_hbm.at[pl.ds(base, _BPW)])


@jax.jit
def _run(t, beta_pad, alpha_pad):
    mesh = plsc.VectorSubcoreMesh(core_axis_name="c", subcore_axis_name="s")
    fn = pl.kernel(
        _gather_body,
        mesh=mesh,
        out_type=(
            jax.ShapeDtypeStruct((_B,), jnp.float32),
            jax.ShapeDtypeStruct((_B,), jnp.float32),
        ),
        scratch_types=[
            pltpu.VMEM((_BPW,), jnp.int32),
            pltpu.VMEM((_TPAD,), jnp.float32),
            pltpu.VMEM((_TPAD,), jnp.float32),
            pltpu.VMEM((_BPW,), jnp.float32),
            pltpu.VMEM((_BPW,), jnp.float32),
        ],
        compiler_params=pltpu.CompilerParams(needs_layout_passes=False),
    )
    return fn(t, beta_pad, alpha_pad)


def kernel(t, beta, alpha_cum):
    t32 = t.astype(jnp.int32)
    beta_pad = jnp.zeros((_TPAD,), jnp.float32).at[:_T].set(beta)
    alpha_pad = jnp.zeros((_TPAD,), jnp.float32).at[:_T].set(alpha_cum)
    return _run(t32, beta_pad, alpha_pad)


# speedup vs baseline: 8.4080x; 1.0100x over previous
import jax
import jax.numpy as jnp
from jax import lax
from jax.experimental import pallas as pl
from jax.experimental.pallas import tpu as pltpu
from jax.experimental.pallas import tpu_sc as plsc

_B = 16384
_T = 1000

_info = plsc.get_sparse_core_info()
_NC, _NS, _L = _info.num_cores, _info.num_subcores, _info.num_lanes
_NW = _NC * _NS
_BPW = _B // _NW


def _gather_body(t_hbm, beta_hbm, alpha_hbm, out_b_hbm, out_a_hbm,
                 idx_v, beta_v, alpha_v, outb_v, outa_v, sem):
    wid = lax.axis_index("s") * _NC + lax.axis_index("c")
    base = wid * _BPW
    c_idx = pltpu.async_copy(t_hbm.at[pl.ds(base, _BPW)], idx_v, sem)
    c_beta = pltpu.async_copy(beta_hbm, beta_v, sem)
    c_alpha = pltpu.async_copy(alpha_hbm, alpha_v, sem)
    c_idx.wait()
    c_beta.wait()
    for i in range(_BPW // _L):
        sl = pl.ds(i * _L, _L)
        outb_v[sl] = plsc.load_gather(beta_v, [idx_v[sl]])
    c_alpha.wait()
    for i in range(_BPW // _L):
        sl = pl.ds(i * _L, _L)
        outa_v[sl] = plsc.load_gather(alpha_v, [idx_v[sl]])
    o_b = pltpu.async_copy(outb_v, out_b_hbm.at[pl.ds(base, _BPW)], sem)
    o_a = pltpu.async_copy(outa_v, out_a_hbm.at[pl.ds(base, _BPW)], sem)
    o_b.wait()
    o_a.wait()


@jax.jit
def _run(t, beta, alpha_cum):
    mesh = plsc.VectorSubcoreMesh(core_axis_name="c", subcore_axis_name="s")
    fn = pl.kernel(
        _gather_body,
        mesh=mesh,
        out_type=(
            jax.ShapeDtypeStruct((_B,), jnp.float32),
            jax.ShapeDtypeStruct((_B,), jnp.float32),
        ),
        scratch_types=[
            pltpu.VMEM((_BPW,), jnp.int32),
            pltpu.VMEM((_T,), jnp.float32),
            pltpu.VMEM((_T,), jnp.float32),
            pltpu.VMEM((_BPW,), jnp.float32),
            pltpu.VMEM((_BPW,), jnp.float32),
            pltpu.SemaphoreType.DMA,
        ],
        compiler_params=pltpu.CompilerParams(needs_layout_passes=False),
    )
    return fn(t, beta, alpha_cum)


def kernel(t, beta, alpha_cum):
    return _run(t.astype(jnp.int32), beta, alpha_cum)


# speedup vs baseline: 8.4305x; 1.0027x over previous
import jax
import jax.numpy as jnp
from jax import lax
from jax.experimental import pallas as pl
from jax.experimental.pallas import tpu as pltpu
from jax.experimental.pallas import tpu_sc as plsc

_B = 16384
_T = 1000

_info = plsc.get_sparse_core_info()
_NC, _NS, _L = _info.num_cores, _info.num_subcores, _info.num_lanes
_NW = _NC * _NS
_BPW = _B // _NW


def _gather_body(t_hbm, beta_hbm, alpha_hbm, out_b_hbm, out_a_hbm,
                 idx_v, beta_v, alpha_v, outb_v, outa_v, sem):
    wid = lax.axis_index("s") * _NC + lax.axis_index("c")
    base = wid * _BPW
    c_idx = pltpu.async_copy(t_hbm.at[pl.ds(base, _BPW)], idx_v, sem)
    c_beta = pltpu.async_copy(beta_hbm, beta_v, sem)
    c_alpha = pltpu.async_copy(alpha_hbm, alpha_v, sem)
    c_idx.wait()
    c_beta.wait()
    c_alpha.wait()
    for i in range(_BPW // _L):
        sl = pl.ds(i * _L, _L)
        idx = idx_v[sl]
        outb_v[sl] = plsc.load_gather(beta_v, [idx])
        outa_v[sl] = plsc.load_gather(alpha_v, [idx])
    o_b = pltpu.async_copy(outb_v, out_b_hbm.at[pl.ds(base, _BPW)], sem)
    o_a = pltpu.async_copy(outa_v, out_a_hbm.at[pl.ds(base, _BPW)], sem)
    o_b.wait()
    o_a.wait()


@jax.jit
def _run(t, beta, alpha_cum):
    mesh = plsc.VectorSubcoreMesh(core_axis_name="c", subcore_axis_name="s")
    fn = pl.kernel(
        _gather_body,
        mesh=mesh,
        out_type=(
            jax.ShapeDtypeStruct((_B,), jnp.float32),
            jax.ShapeDtypeStruct((_B,), jnp.float32),
        ),
        scratch_types=[
            pltpu.VMEM((_BPW,), jnp.int32),
            pltpu.VMEM((_T,), jnp.float32),
            pltpu.VMEM((_T,), jnp.float32),
            pltpu.VMEM((_BPW,), jnp.float32),
            pltpu.VMEM((_BPW,), jnp.float32),
            pltpu.SemaphoreType.DMA,
        ],
        compiler_params=pltpu.CompilerParams(needs_layout_passes=False),
    )
    return fn(t, beta, alpha_cum)


def kernel(t, beta, alpha_cum):
    return _run(t.astype(jnp.int32), beta, alpha_cum)
